# baseline (speedup 1.0000x reference)
import functools

import jax
import jax.numpy as jnp
from jax import lax
from jax.experimental import pallas as pl
from jax.experimental.pallas import tpu as pltpu
from jax.experimental.pallas import tpu_sc as plsc

E = 8
TOP_K = 2
H = 1024
FFN = 512
SFFN = 2048
T = 2048

TMA = 256
TM = 256
SEG = T // 2
MAXI = SEG // TM

NC = 2
NS = 16
TW = T // (NC * NS)
CH = 32
NCH = 2 * TW // CH
L = 16


def _router_shared_body(x_ref, rw_ref, shg_ref, shu_ref, shd_ref,
                        xw_ref, idx_ref, sh_ref, bases_ref, cnt_ref, run_ref):
    t = pl.program_id(0)
    x = x_ref[...]

    logits = lax.dot_general(x, rw_ref[...], (((1,), (1,)), ((), ())),
                             preferred_element_type=jnp.float32)
    col = lax.broadcasted_iota(jnp.int32, (TMA, E), 1)
    m1 = jnp.max(logits, axis=1, keepdims=True)
    a1 = jnp.min(jnp.where(logits == m1, col, E), axis=1, keepdims=True)
    logits2 = jnp.where(col == a1, -jnp.inf, logits)
    m2 = jnp.max(logits2, axis=1, keepdims=True)
    a2 = jnp.min(jnp.where(logits2 == m2, col, E), axis=1, keepdims=True)
    w1 = jax.nn.sigmoid(m1)
    w2 = jax.nn.sigmoid(m2)

    xw_ref[...] = jnp.stack([x * w1, x * w2], axis=0)
    idx_ref[...] = jnp.concatenate(
        [a1.reshape(1, TMA), a2.reshape(1, TMA)], axis=0)

    keep = (col == a1) | (col == a2)
    keepi = jnp.concatenate(
        [keep.astype(jnp.int32), jnp.zeros((TMA, L - E), jnp.int32)], axis=1)
    hist = jnp.sum(keepi.reshape(TMA // TW, TW, L), axis=1)
    run = jnp.where(t % (SEG // TMA) == 0, 0, run_ref[...])
    rows = []
    for c in range(TMA // TW):
        rows.append(run)
        run = run + hist[c:c + 1, :]
    bases_ref[...] = jnp.concatenate(rows, axis=0).reshape(1, TMA // TW, L)
    cnt_ref[...] = run.reshape(1, 1, L)
    run_ref[...] = run

    gsh = lax.dot_general(x, shg_ref[...], (((1,), (1,)), ((), ())),
                          preferred_element_type=jnp.float32)
    ush = lax.dot_general(x, shu_ref[...], (((1,), (1,)), ((), ())),
                          preferred_element_type=jnp.float32)
    hsh = ush * (gsh * jax.nn.sigmoid(gsh))
    sh_ref[...] = lax.dot_general(hsh, shd_ref[...], (((1,), (1,)), ((), ())),
                                  preferred_element_type=jnp.float32)


def _router_shared(hidden, router_w, sh_gate_w, sh_up_w, sh_down_w):
    return pl.pallas_call(
        _router_shared_body,
        grid=(T // TMA,),
        in_specs=[
            pl.BlockSpec((TMA, H), lambda t: (t, 0)),
            pl.BlockSpec((E, H), lambda t: (0, 0)),
            pl.BlockSpec((SFFN, H), lambda t: (0, 0)),
            pl.BlockSpec((SFFN, H), lambda t: (0, 0)),
            pl.BlockSpec((H, SFFN), lambda t: (0, 0)),
        ],
        out_specs=[
            pl.BlockSpec((2, TMA, H), lambda t: (0, t, 0)),
            pl.BlockSpec((2, TMA), lambda t: (0, t)),
            pl.BlockSpec((TMA, H), lambda t: (t, 0)),
            pl.BlockSpec((1, TMA // TW, L), lambda t: (t, 0, 0)),
            pl.BlockSpec((1, 1, L), lambda t: (t // (SEG // TMA), 0, 0)),
        ],
        out_shape=[
            jax.ShapeDtypeStruct((2, T, H), jnp.float32),
            jax.ShapeDtypeStruct((2, T), jnp.int32),
            jax.ShapeDtypeStruct((T, H), jnp.float32),
            jax.ShapeDtypeStruct((T // TMA, TMA // TW, L), jnp.int32),
            jax.ShapeDtypeStruct((NC, 1, L), jnp.int32),
        ],
        scratch_shapes=[pltpu.VMEM((1, L), jnp.int32)],
    )(hidden, router_w, sh_gate_w, sh_up_w, sh_down_w)


def _dispatch_body(idxt_hbm, xw_hbm, bases_hbm, xbuf_hbm, dst_hbm,
                   idxv, basev, dst2d, rows):
    sc = lax.axis_index("c")
    sub = lax.axis_index("s")
    t0 = sc * SEG + sub * TW

    pltpu.sync_copy(idxt_hbm.at[0, pl.ds(t0, TW)], idxv.at[pl.ds(0, TW)])
    pltpu.sync_copy(idxt_hbm.at[1, pl.ds(t0, TW)], idxv.at[pl.ds(TW, TW)])
    w = sc * NS + sub
    pltpu.sync_copy(bases_hbm.at[w // (TMA // TW), w % (TMA // TW)], basev)

    ii = lax.broadcasted_iota(jnp.int32, (L,), 0)
    zero = jnp.zeros((L,), jnp.int32)

    def bc(s):
        return jnp.broadcast_to(s, (L,))

    seg0 = bc(sc * SEG)
    off = basev[...]
    for j in range(2 * TW // L):
        v = idxv[pl.ds(j * L, L)]
        dstj = zero
        for e in range(E):
            ev = jnp.full((L,), e, jnp.int32)
            m = v == ev
            lane_e = ii == ev
            pfx = plsc.cumsum(jnp.where(m, jnp.ones((L,), jnp.int32), zero))
            off_e = jnp.sum(jnp.where(lane_e, off, zero))
            pos = bc(off_e) + pfx - jnp.ones((L,), jnp.int32)
            dstj = jnp.where(m, jnp.full((L,), e * T, jnp.int32) + seg0 + pos,
                             dstj)
            off = off + jnp.where(lane_e, bc(pfx[L - 1]), zero)
        c, h = j // 2, j % 2
        dst2d[c, pl.ds(h * L, L)] = dstj

    pltpu.sync_copy(dst2d, dst_hbm.at[sc * NS + sub])

    for c in range(NCH):
        k = c // 2
        r0 = t0 + (c % 2) * CH
        pltpu.sync_copy(xw_hbm.at[k, pl.ds(r0, CH)], rows)
        pltpu.sync_copy(rows, xbuf_hbm.at[dst2d.at[c]])


def _dispatch(idxt, xw, bases):
    mesh = plsc.VectorSubcoreMesh(core_axis_name="c", subcore_axis_name="s")
    kern = pl.kernel(
        _dispatch_body,
        out_type=[
            jax.ShapeDtypeStruct((E * T, H), jnp.float32),
            jax.ShapeDtypeStruct((NC * NS, NCH, CH), jnp.int32),
        ],
        mesh=mesh,
        scratch_types=[
            pltpu.VMEM((2 * TW,), jnp.int32),
            pltpu.VMEM((L,), jnp.int32),
            pltpu.VMEM((NCH, CH), jnp.int32),
            pltpu.VMEM((CH, H), jnp.float32),
        ],
        compiler_params=pltpu.CompilerParams(needs_layout_passes=False),
    )
    return kern(idxt, xw, bases)


def _group_mm_body(cnt_ref, x_ref, gu_ref, dn_ref, y_ref):
    s = pl.program_id(1)
    e = pl.program_id(0)
    i = pl.program_id(2)

    @pl.when(i * TM < cnt_ref[s, 0, e])
    def _():
        x = x_ref[...]
        gu = jnp.dot(x, gu_ref[0], preferred_element_type=jnp.float32)
        g = gu[:, :FFN]
        u = gu[:, FFN:]
        h = u * (g * jax.nn.sigmoid(g))
        y_ref[...] = jnp.dot(h, dn_ref[0], preferred_element_type=jnp.float32)


def _block_of(e, s, i, cnt_ref):
    c = cnt_ref[s, 0, e]
    nt = (c + (TM - 1)) // TM
    ieff = jnp.minimum(i, jnp.maximum(nt - 1, 0))
    return e * (T // TM) + s * MAXI + ieff


def _group_mm(counts, xbuf, gate_up_proj, down_proj):
    grid = (E, NC, MAXI)
    return pl.pallas_call(
        _group_mm_body,
        grid_spec=pltpu.PrefetchScalarGridSpec(
            num_scalar_prefetch=1,
            grid=grid,
            in_specs=[
                pl.BlockSpec((TM, H), lambda e, s, i, cnt: (_block_of(e, s, i, cnt), 0)),
                pl.BlockSpec((1, H, 2 * FFN), lambda e, s, i, cnt: (e, 0, 0)),
                pl.BlockSpec((1, FFN, H), lambda e, s, i, cnt: (e, 0, 0)),
            ],
            out_specs=pl.BlockSpec((TM, H), lambda e, s, i, cnt: (_block_of(e, s, i, cnt), 0)),
        ),
        out_shape=jax.ShapeDtypeStruct((E * T, H), jnp.float32),
    )(counts, xbuf, gate_up_proj, down_proj)


def _combine_body(y_hbm, sh_hbm, dst_hbm, out_hbm, dstv, y0, y1, shrows, orows):
    sc = lax.axis_index("c")
    sub = lax.axis_index("s")
    t0 = sc * SEG + sub * TW

    pltpu.sync_copy(dst_hbm.at[sc * NS + sub], dstv)

    for g in range(TW // L):
        c0, h = g // 2, (g % 2) * L
        pltpu.sync_copy(y_hbm.at[dstv.at[c0, pl.ds(h, L)]], y0)
        pltpu.sync_copy(y_hbm.at[dstv.at[2 + c0, pl.ds(h, L)]], y1)
        pltpu.sync_copy(sh_hbm.at[pl.ds(t0 + g * L, L)], shrows)

        def body(m, _):
            dsm = pl.ds(m * L, L)
            for r in range(L):
                orows[r, dsm] = shrows[r, dsm] + y0[r, dsm] + y1[r, dsm]
            return 0

        lax.fori_loop(0, H // L, body, 0)
        pltpu.sync_copy(orows, out_hbm.at[pl.ds(t0 + g * L, L)])


def _combine(y, shared, dst):
    mesh = plsc.VectorSubcoreMesh(core_axis_name="c", subcore_axis_name="s")
    kern = pl.kernel(
        _combine_body,
        out_type=jax.ShapeDtypeStruct((T, H), jnp.float32),
        mesh=mesh,
        scratch_types=[
            pltpu.VMEM((NCH, CH), jnp.int32),
            pltpu.VMEM((L, H), jnp.float32),
            pltpu.VMEM((L, H), jnp.float32),
            pltpu.VMEM((L, H), jnp.float32),
            pltpu.VMEM((L, H), jnp.float32),
        ],
        compiler_params=pltpu.CompilerParams(needs_layout_passes=False),
    )
    return kern(y, shared, dst)


@jax.jit
def _moe(hidden, router_w, gate_up_proj, down_proj, sh_gate_w, sh_up_w, sh_down_w):
    xw, idxt, shared, bases, counts = _router_shared(
        hidden, router_w, sh_gate_w, sh_up_w, sh_down_w)
    xbuf, dst = _dispatch(idxt, xw, bases)
    y = _group_mm(counts, xbuf, gate_up_proj, down_proj)
    return _combine(y, shared, dst)


def kernel(hidden_states, router_w, gate_up_proj, down_proj, sh_gate_w, sh_up_w, sh_down_w):
    B, S, Hd = hidden_states.shape
    hidden = hidden_states.reshape(-1, Hd)
    out = _moe(hidden, router_w, gate_up_proj, down_proj, sh_gate_w, sh_up_w, sh_down_w)
    return out.reshape(B, S, Hd)


# speedup vs baseline: 1.1024x; 1.1024x over previous
import functools

import jax
import jax.numpy as jnp
from jax import lax
from jax.experimental import pallas as pl
from jax.experimental.pallas import tpu as pltpu
from jax.experimental.pallas import tpu_sc as plsc

E = 8
TOP_K = 2
H = 1024
FFN = 512
SFFN = 2048
T = 2048

TMA = 256
TM = 256
SEG = T // 2
MAXI = SEG // TM

NC = 2
NS = 16
TW = T // (NC * NS)
CH = 32
NCH = 2 * TW // CH
L = 16


def _router_body(x_ref, rw_ref, xw_ref, idx_ref, bases_ref, cnt_ref, run_ref):
    t = pl.program_id(0)
    x = x_ref[...]

    logits = lax.dot_general(x, rw_ref[...], (((1,), (1,)), ((), ())),
                             preferred_element_type=jnp.float32)
    col = lax.broadcasted_iota(jnp.int32, (TMA, E), 1)
    m1 = jnp.max(logits, axis=1, keepdims=True)
    a1 = jnp.min(jnp.where(logits == m1, col, E), axis=1, keepdims=True)
    logits2 = jnp.where(col == a1, -jnp.inf, logits)
    m2 = jnp.max(logits2, axis=1, keepdims=True)
    a2 = jnp.min(jnp.where(logits2 == m2, col, E), axis=1, keepdims=True)
    w1 = jax.nn.sigmoid(m1)
    w2 = jax.nn.sigmoid(m2)

    xw_ref[...] = jnp.stack([x * w1, x * w2], axis=0)
    idx_ref[...] = jnp.concatenate(
        [a1.reshape(1, TMA), a2.reshape(1, TMA)], axis=0)

    keep = (col == a1) | (col == a2)
    keepi = jnp.concatenate(
        [keep.astype(jnp.int32), jnp.zeros((TMA, L - E), jnp.int32)], axis=1)
    hist = jnp.sum(keepi.reshape(TMA // TW, TW, L), axis=1)
    run = jnp.where(t % (SEG // TMA) == 0, 0, run_ref[...])
    rows = []
    for c in range(TMA // TW):
        rows.append(run)
        run = run + hist[c:c + 1, :]
    bases_ref[...] = jnp.concatenate(rows, axis=0).reshape(1, TMA // TW, L)
    cnt_ref[...] = run.reshape(1, 1, L)
    run_ref[...] = run


def _router(hidden, router_w):
    return pl.pallas_call(
        _router_body,
        grid=(T // TMA,),
        in_specs=[
            pl.BlockSpec((TMA, H), lambda t: (t, 0)),
            pl.BlockSpec((E, H), lambda t: (0, 0)),
        ],
        out_specs=[
            pl.BlockSpec((2, TMA, H), lambda t: (0, t, 0)),
            pl.BlockSpec((2, TMA), lambda t: (0, t)),
            pl.BlockSpec((1, TMA // TW, L), lambda t: (t, 0, 0)),
            pl.BlockSpec((1, 1, L), lambda t: (t // (SEG // TMA), 0, 0)),
        ],
        out_shape=[
            jax.ShapeDtypeStruct((2, T, H), jnp.float32),
            jax.ShapeDtypeStruct((2, T), jnp.int32),
            jax.ShapeDtypeStruct((T // TMA, TMA // TW, L), jnp.int32),
            jax.ShapeDtypeStruct((NC, 1, L), jnp.int32),
        ],
        scratch_shapes=[pltpu.VMEM((1, L), jnp.int32)],
    )(hidden, router_w)


def _shared_body(x_ref, shg_ref, shu_ref, shd_ref, sh_ref):
    x = x_ref[...]
    gsh = lax.dot_general(x, shg_ref[...], (((1,), (1,)), ((), ())),
                          preferred_element_type=jnp.float32)
    ush = lax.dot_general(x, shu_ref[...], (((1,), (1,)), ((), ())),
                          preferred_element_type=jnp.float32)
    hsh = ush * (gsh * jax.nn.sigmoid(gsh))
    sh_ref[...] = lax.dot_general(hsh, shd_ref[...], (((1,), (1,)), ((), ())),
                                  preferred_element_type=jnp.float32)


def _shared(hidden, sh_gate_w, sh_up_w, sh_down_w):
    return pl.pallas_call(
        _shared_body,
        grid=(T // TMA,),
        in_specs=[
            pl.BlockSpec((TMA, H), lambda t: (t, 0)),
            pl.BlockSpec((SFFN, H), lambda t: (0, 0)),
            pl.BlockSpec((SFFN, H), lambda t: (0, 0)),
            pl.BlockSpec((H, SFFN), lambda t: (0, 0)),
        ],
        out_specs=pl.BlockSpec((TMA, H), lambda t: (t, 0)),
        out_shape=jax.ShapeDtypeStruct((T, H), jnp.float32),
    )(hidden, sh_gate_w, sh_up_w, sh_down_w)


def _dispatch_body(idxt_hbm, xw_hbm, bases_hbm, xbuf_hbm, dst_hbm,
                   idxv, basev, dst2d, rows, rows_b, sem_a, sem_b):
    sc = lax.axis_index("c")
    sub = lax.axis_index("s")
    t0 = sc * SEG + sub * TW

    pltpu.sync_copy(idxt_hbm.at[0, pl.ds(t0, TW)], idxv.at[pl.ds(0, TW)])
    pltpu.sync_copy(idxt_hbm.at[1, pl.ds(t0, TW)], idxv.at[pl.ds(TW, TW)])
    w = sc * NS + sub
    pltpu.sync_copy(bases_hbm.at[w // (TMA // TW), w % (TMA // TW)], basev)

    ii = lax.broadcasted_iota(jnp.int32, (L,), 0)
    zero = jnp.zeros((L,), jnp.int32)

    def bc(s):
        return jnp.broadcast_to(s, (L,))

    seg0 = bc(sc * SEG)
    off = basev[...]
    for j in range(2 * TW // L):
        v = idxv[pl.ds(j * L, L)]
        dstj = zero
        for e in range(E):
            ev = jnp.full((L,), e, jnp.int32)
            m = v == ev
            lane_e = ii == ev
            pfx = plsc.cumsum(jnp.where(m, jnp.ones((L,), jnp.int32), zero))
            off_e = jnp.sum(jnp.where(lane_e, off, zero))
            pos = bc(off_e) + pfx - jnp.ones((L,), jnp.int32)
            dstj = jnp.where(m, jnp.full((L,), e * T, jnp.int32) + seg0 + pos,
                             dstj)
            off = off + jnp.where(lane_e, bc(pfx[L - 1]), zero)
        c, h = j // 2, j % 2
        dst2d[c, pl.ds(h * L, L)] = dstj

    pltpu.sync_copy(dst2d, dst_hbm.at[w])

    def rd(c, buf, sem):
        k, r0 = c // 2, t0 + (c % 2) * CH
        return pltpu.async_copy(xw_hbm.at[k, pl.ds(r0, CH)], buf, sem)

    def st(c, buf, sem):
        return pltpu.async_copy(buf, xbuf_hbm.at[dst2d.at[c]], sem)

    r0h = rd(0, rows, sem_a)
    r1h = rd(1, rows_b, sem_b)
    r0h.wait()
    s0 = st(0, rows, sem_a)
    r1h.wait()
    s1 = st(1, rows_b, sem_b)
    s0.wait()
    r2h = rd(2, rows, sem_a)
    s1.wait()
    r3h = rd(3, rows_b, sem_b)
    r2h.wait()
    s2 = st(2, rows, sem_a)
    r3h.wait()
    s3 = st(3, rows_b, sem_b)
    s2.wait()
    s3.wait()


def _dispatch(idxt, xw, bases):
    mesh = plsc.VectorSubcoreMesh(core_axis_name="c", subcore_axis_name="s")
    kern = pl.kernel(
        _dispatch_body,
        out_type=[
            jax.ShapeDtypeStruct((E * T, H), jnp.float32),
            jax.ShapeDtypeStruct((NC * NS, NCH, CH), jnp.int32),
        ],
        mesh=mesh,
        scratch_types=[
            pltpu.VMEM((2 * TW,), jnp.int32),
            pltpu.VMEM((L,), jnp.int32),
            pltpu.VMEM((NCH, CH), jnp.int32),
            pltpu.VMEM((CH, H), jnp.float32),
            pltpu.VMEM((CH, H), jnp.float32),
            pltpu.SemaphoreType.DMA,
            pltpu.SemaphoreType.DMA,
        ],
        compiler_params=pltpu.CompilerParams(needs_layout_passes=False),
    )
    return kern(idxt, xw, bases)


def _group_mm_body(cnt_ref, x_ref, gu_ref, dn_ref, y_ref):
    s = pl.program_id(1)
    e = pl.program_id(0)
    i = pl.program_id(2)

    @pl.when(i * TM < cnt_ref[s, 0, e])
    def _():
        x = x_ref[...]
        gu = jnp.dot(x, gu_ref[0], preferred_element_type=jnp.float32)
        g = gu[:, :FFN]
        u = gu[:, FFN:]
        h = u * (g * jax.nn.sigmoid(g))
        y_ref[...] = jnp.dot(h, dn_ref[0], preferred_element_type=jnp.float32)


def _block_of(e, s, i, cnt_ref):
    c = cnt_ref[s, 0, e]
    nt = (c + (TM - 1)) // TM
    ieff = jnp.minimum(i, jnp.maximum(nt - 1, 0))
    return e * (T // TM) + s * MAXI + ieff


def _group_mm(counts, xbuf, gate_up_proj, down_proj):
    grid = (E, NC, MAXI)
    return pl.pallas_call(
        _group_mm_body,
        grid_spec=pltpu.PrefetchScalarGridSpec(
            num_scalar_prefetch=1,
            grid=grid,
            in_specs=[
                pl.BlockSpec((TM, H), lambda e, s, i, cnt: (_block_of(e, s, i, cnt), 0)),
                pl.BlockSpec((1, H, 2 * FFN), lambda e, s, i, cnt: (e, 0, 0)),
                pl.BlockSpec((1, FFN, H), lambda e, s, i, cnt: (e, 0, 0)),
            ],
            out_specs=pl.BlockSpec((TM, H), lambda e, s, i, cnt: (_block_of(e, s, i, cnt), 0)),
        ),
        out_shape=jax.ShapeDtypeStruct((E * T, H), jnp.float32),
    )(counts, xbuf, gate_up_proj, down_proj)


def _combine_body(y_hbm, sh_hbm, dst_hbm, out_hbm, dstv,
                  y0a, y1a, sha, y0b, y1b, shb, orows, sem_a, sem_b):
    sc = lax.axis_index("c")
    sub = lax.axis_index("s")
    t0 = sc * SEG + sub * TW

    pltpu.sync_copy(dst_hbm.at[sc * NS + sub], dstv)

    bufs = [(y0a, y1a, sha, sem_a), (y0b, y1b, shb, sem_b)]

    def fire(g):
        y0, y1, sh, sem = bufs[g % 2]
        c0, h = g // 2, (g % 2) * L
        return (
            pltpu.async_copy(y_hbm.at[dstv.at[c0, pl.ds(h, L)]], y0, sem),
            pltpu.async_copy(y_hbm.at[dstv.at[2 + c0, pl.ds(h, L)]], y1, sem),
            pltpu.async_copy(sh_hbm.at[pl.ds(t0 + g * L, L)], sh, sem),
        )

    pend = fire(0)
    for g in range(TW // L):
        for hnd in pend:
            hnd.wait()
        y0, y1, sh, _ = bufs[g % 2]
        if g + 1 < TW // L:
            pend = fire(g + 1)

        def body(m, _):
            dsm = pl.ds(m * L, L)
            for r in range(L):
                orows[r, dsm] = sh[r, dsm] + y0[r, dsm] + y1[r, dsm]
            return 0

        lax.fori_loop(0, H // L, body, 0)
        pltpu.sync_copy(orows, out_hbm.at[pl.ds(t0 + g * L, L)])


def _combine(y, shared, dst):
    mesh = plsc.VectorSubcoreMesh(core_axis_name="c", subcore_axis_name="s")
    kern = pl.kernel(
        _combine_body,
        out_type=jax.ShapeDtypeStruct((T, H), jnp.float32),
        mesh=mesh,
        scratch_types=[
            pltpu.VMEM((NCH, CH), jnp.int32),
            pltpu.VMEM((L, H), jnp.float32),
            pltpu.VMEM((L, H), jnp.float32),
            pltpu.VMEM((L, H), jnp.float32),
            pltpu.VMEM((L, H), jnp.float32),
            pltpu.VMEM((L, H), jnp.float32),
            pltpu.VMEM((L, H), jnp.float32),
            pltpu.VMEM((L, H), jnp.float32),
            pltpu.SemaphoreType.DMA,
            pltpu.SemaphoreType.DMA,
        ],
        compiler_params=pltpu.CompilerParams(needs_layout_passes=False),
    )
    return kern(y, shared, dst)


@jax.jit
def _moe(hidden, router_w, gate_up_proj, down_proj, sh_gate_w, sh_up_w, sh_down_w):
    xw, idxt, bases, counts = _router(hidden, router_w)
    xbuf, dst = _dispatch(idxt, xw, bases)
    shared = _shared(hidden, sh_gate_w, sh_up_w, sh_down_w)
    y = _group_mm(counts, xbuf, gate_up_proj, down_proj)
    return _combine(y, shared, dst)


def kernel(hidden_states, router_w, gate_up_proj, down_proj, sh_gate_w, sh_up_w, sh_down_w):
    B, S, Hd = hidden_states.shape
    hidden = hidden_states.reshape(-1, Hd)
    out = _moe(hidden, router_w, gate_up_proj, down_proj, sh_gate_w, sh_up_w, sh_down_w)
    return out.reshape(B, S, Hd)
